# Initial kernel scaffold; baseline (speedup 1.0000x reference)
import jax
import jax.numpy as jnp
from jax.experimental import pallas as pl


def kernel(seqs, query_tok, embed, W1, b1, W2, b2):
    raise NotImplementedError("write your pallas kernel here")


# speedup vs baseline: 14.2372x; 14.2372x over previous
import functools

import jax
import jax.numpy as jnp
from jax.experimental import pallas as pl
from jax.experimental.pallas import tpu as pltpu

H = 64
MEM = 8
SEQ = 64
VOCAB = 66
NOUT = 64
B = 16384
BT = 2048


def _fused_body(idx_ref, emb_ref, w1_ref, b1_ref, w2_ref, b2_ref, out_ref):
    emb = emb_ref[...]
    w1 = w1_ref[...]
    dn = (((1,), (1,)), ((), ()))
    eq = jax.lax.dot_general(emb, w1[:, :H], dn,
                             preferred_element_type=jnp.float32) + b1_ref[...]
    em = jax.lax.dot_general(emb, w1[:, H:], dn,
                             preferred_element_type=jnp.float32) * (1.0 / MEM)

    idx = idx_ref[...]
    iota = jax.lax.broadcasted_iota(jnp.int32, (1, VOCAB), 1)
    q1 = (idx[:, 0:1] == iota).astype(jnp.float32)
    cnt = (idx[:, 1:2] == iota).astype(jnp.float32)
    for j in range(2, MEM + 1):
        cnt += (idx[:, j:j + 1] == iota).astype(jnp.float32)

    pre = (jnp.dot(q1, eq, preferred_element_type=jnp.float32)
           + jnp.dot(cnt, em, preferred_element_type=jnp.float32))
    h = jnp.maximum(pre, 0.0)
    out_ref[...] = jax.lax.dot_general(h, w2_ref[...], dn,
                                       preferred_element_type=jnp.float32) + b2_ref[...]


@jax.jit
def _fused(idx_all, embed, W1, b1, W2, b2):
    grid = (B // BT,)
    return pl.pallas_call(
        _fused_body,
        grid=grid,
        in_specs=[
            pl.BlockSpec((BT, MEM + 1), lambda i: (i, 0)),
            pl.BlockSpec((VOCAB, H), lambda i: (0, 0)),
            pl.BlockSpec((H, 2 * H), lambda i: (0, 0)),
            pl.BlockSpec((1, H), lambda i: (0, 0)),
            pl.BlockSpec((NOUT, H), lambda i: (0, 0)),
            pl.BlockSpec((1, NOUT), lambda i: (0, 0)),
        ],
        out_specs=pl.BlockSpec((BT, NOUT), lambda i: (i, 0)),
        out_shape=jax.ShapeDtypeStruct((B, NOUT), jnp.float32),
    )(idx_all, embed, W1, b1, W2, b2)


def kernel(seqs, query_tok, embed, W1, b1, W2, b2):
    start = SEQ - 1 - MEM
    mem_tokens = seqs[:, start:SEQ - 1]
    idx_all = jnp.concatenate(
        [query_tok[:, None], mem_tokens], axis=1).astype(jnp.int32)
    return _fused(idx_all, embed, W1,
                  b1.reshape(1, H), W2, b2.reshape(1, NOUT))
